# baseline (speedup 1.0000x reference)
import functools

import jax
import jax.numpy as jnp
from jax import lax
from jax.experimental import pallas as pl
from jax.experimental.pallas import tpu as pltpu

PARTIAL_DATA_THRESHOLD = 0.01
WINDOW = 0.100001

TBLK = 512
SBLK = 512


def _chamfer_kernel(clo_ref, chi_ref, t_ref, s_ref, out_ref):
    i = pl.program_id(0)

    @pl.when(i == 0)
    def _init_out():
        out_ref[:, :] = jnp.zeros((1, 1), dtype=jnp.float32)

    tx = t_ref[:, 0:1]
    ty = t_ref[:, 1:2]
    tz = t_ref[:, 2:3]

    def body(c, dmin):
        chunk = s_ref[c]
        sx = chunk[0:1, :]
        sy = chunk[1:2, :]
        sz = chunk[2:3, :]
        dx = tx - sx
        dy = ty - sy
        dz = tz - sz
        d = dx * dx + dy * dy + dz * dz
        return jnp.minimum(dmin, jnp.min(d, axis=1, keepdims=True))

    dmin0 = jnp.full((TBLK, 1), jnp.inf, dtype=jnp.float32)
    dmin = lax.fori_loop(clo_ref[i], chi_ref[i], body, dmin0)
    contrib = jnp.sum(
        jnp.where(dmin < PARTIAL_DATA_THRESHOLD, dmin, 0.0),
        axis=(0, 1), keepdims=True)
    out_ref[:, :] += contrib


def kernel(scan_vertices, template_vertices):
    n = scan_vertices.shape[0]
    m = template_vertices.shape[0]
    n_tblk = m // TBLK
    n_schunk = n // SBLK

    sxs, sys_, szs = lax.sort(
        [scan_vertices[:, 0], scan_vertices[:, 1], scan_vertices[:, 2]],
        num_keys=1)
    txs, tys, tzs = lax.sort(
        [template_vertices[:, 0], template_vertices[:, 1],
         template_vertices[:, 2]],
        num_keys=1)

    scan_s = jnp.stack([sxs, sys_, szs]).reshape(3, n_schunk, SBLK)
    scan_s = jnp.transpose(scan_s, (1, 0, 2))
    temp_s = jnp.stack([txs, tys, tzs], axis=-1)

    tb = txs.reshape(n_tblk, TBLK)
    lo = jnp.searchsorted(sxs, tb[:, 0] - WINDOW, side="left")
    hi = jnp.searchsorted(sxs, tb[:, -1] + WINDOW, side="right")
    clo = (lo // SBLK).astype(jnp.int32)
    chi = ((hi + SBLK - 1) // SBLK).astype(jnp.int32)

    out = pl.pallas_call(
        _chamfer_kernel,
        grid=(n_tblk,),
        in_specs=[
            pl.BlockSpec(memory_space=pltpu.SMEM),
            pl.BlockSpec(memory_space=pltpu.SMEM),
            pl.BlockSpec((TBLK, 3), lambda i: (i, 0)),
            pl.BlockSpec((n_schunk, 3, SBLK), lambda i: (0, 0, 0)),
        ],
        out_specs=pl.BlockSpec((1, 1), lambda i: (0, 0)),
        out_shape=jax.ShapeDtypeStruct((1, 1), jnp.float32),
    )(clo, chi, temp_s, scan_s)
    return out[0, 0]


# speedup vs baseline: 1.0076x; 1.0076x over previous
import functools

import jax
import jax.numpy as jnp
from jax import lax
from jax.experimental import pallas as pl
from jax.experimental.pallas import tpu as pltpu

PARTIAL_DATA_THRESHOLD = 0.01
WINDOW = 0.100001

TBLK = 512
SBLK = 1024
GRP = 128


def _chamfer_kernel(clo_ref, chi_ref, t_ref, s_ref, out_ref):
    i = pl.program_id(0)

    @pl.when(i == 0)
    def _init_out():
        out_ref[:, :] = jnp.zeros((1, 1), dtype=jnp.float32)

    tx = t_ref[:, 0:1]
    ty = t_ref[:, 1:2]
    tz = t_ref[:, 2:3]

    def body(c, dmin):
        chunk = s_ref[c]
        for g in range(SBLK // GRP):
            sx = chunk[0:1, g * GRP:(g + 1) * GRP]
            sy = chunk[1:2, g * GRP:(g + 1) * GRP]
            sz = chunk[2:3, g * GRP:(g + 1) * GRP]
            dx = tx - sx
            dy = ty - sy
            dz = tz - sz
            d = dx * dx + dy * dy + dz * dz
            dmin = jnp.minimum(dmin, jnp.min(d, axis=1, keepdims=True))
        return dmin

    dmin0 = jnp.full((TBLK, 1), jnp.inf, dtype=jnp.float32)
    dmin = lax.fori_loop(clo_ref[i], chi_ref[i], body, dmin0)
    contrib = jnp.sum(
        jnp.where(dmin < PARTIAL_DATA_THRESHOLD, dmin, 0.0),
        axis=(0, 1), keepdims=True)
    out_ref[:, :] += contrib


def kernel(scan_vertices, template_vertices):
    n = scan_vertices.shape[0]
    m = template_vertices.shape[0]
    n_tblk = m // TBLK
    n_schunk = n // SBLK

    sxs, sys_, szs = lax.sort(
        [scan_vertices[:, 0], scan_vertices[:, 1], scan_vertices[:, 2]],
        num_keys=1)
    txs, tys, tzs = lax.sort(
        [template_vertices[:, 0], template_vertices[:, 1],
         template_vertices[:, 2]],
        num_keys=1)

    scan_s = jnp.stack([sxs, sys_, szs]).reshape(3, n_schunk, SBLK)
    scan_s = jnp.transpose(scan_s, (1, 0, 2))
    temp_s = jnp.stack([txs, tys, tzs], axis=-1)

    tb = txs.reshape(n_tblk, TBLK)
    lo = jnp.searchsorted(sxs, tb[:, 0] - WINDOW, side="left")
    hi = jnp.searchsorted(sxs, tb[:, -1] + WINDOW, side="right")
    clo = (lo // SBLK).astype(jnp.int32)
    chi = ((hi + SBLK - 1) // SBLK).astype(jnp.int32)

    out = pl.pallas_call(
        _chamfer_kernel,
        grid=(n_tblk,),
        in_specs=[
            pl.BlockSpec(memory_space=pltpu.SMEM),
            pl.BlockSpec(memory_space=pltpu.SMEM),
            pl.BlockSpec((TBLK, 3), lambda i: (i, 0)),
            pl.BlockSpec((n_schunk, 3, SBLK), lambda i: (0, 0, 0)),
        ],
        out_specs=pl.BlockSpec((1, 1), lambda i: (0, 0)),
        out_shape=jax.ShapeDtypeStruct((1, 1), jnp.float32),
    )(clo, chi, temp_s, scan_s)
    return out[0, 0]


# speedup vs baseline: 1.0871x; 1.0789x over previous
import functools

import jax
import jax.numpy as jnp
from jax import lax
from jax.experimental import pallas as pl
from jax.experimental.pallas import tpu as pltpu

PARTIAL_DATA_THRESHOLD = 0.01
WINDOW = 0.100001

TBLK = 512
SBLK = 1024
GRP = 256


def _chamfer_kernel(clo_ref, chi_ref, t_ref, s_ref, out_ref):
    i = pl.program_id(0)

    @pl.when(i == 0)
    def _init_out():
        out_ref[:, :] = jnp.zeros((1, 1), dtype=jnp.float32)

    tx = t_ref[:, 0:1]
    ty = t_ref[:, 1:2]
    tz = t_ref[:, 2:3]

    def body(c, dmin):
        chunk = s_ref[c]
        for g in range(SBLK // GRP):
            sx = chunk[0:1, g * GRP:(g + 1) * GRP]
            sy = chunk[1:2, g * GRP:(g + 1) * GRP]
            sz = chunk[2:3, g * GRP:(g + 1) * GRP]
            dx = tx - sx
            dy = ty - sy
            dz = tz - sz
            d = dx * dx + dy * dy + dz * dz
            dmin = jnp.minimum(dmin, jnp.min(d, axis=1, keepdims=True))
        return dmin

    dmin0 = jnp.full((TBLK, 1), jnp.inf, dtype=jnp.float32)
    dmin = lax.fori_loop(clo_ref[i], chi_ref[i], body, dmin0)
    contrib = jnp.sum(
        jnp.where(dmin < PARTIAL_DATA_THRESHOLD, dmin, 0.0),
        axis=(0, 1), keepdims=True)
    out_ref[:, :] += contrib


def kernel(scan_vertices, template_vertices):
    n = scan_vertices.shape[0]
    m = template_vertices.shape[0]
    n_tblk = m // TBLK
    n_schunk = n // SBLK

    sxs, sys_, szs = lax.sort(
        [scan_vertices[:, 0], scan_vertices[:, 1], scan_vertices[:, 2]],
        num_keys=1)
    txs, tys, tzs = lax.sort(
        [template_vertices[:, 0], template_vertices[:, 1],
         template_vertices[:, 2]],
        num_keys=1)

    scan_s = jnp.stack([sxs, sys_, szs]).reshape(3, n_schunk, SBLK)
    scan_s = jnp.transpose(scan_s, (1, 0, 2))
    temp_s = jnp.stack([txs, tys, tzs], axis=-1)

    tb = txs.reshape(n_tblk, TBLK)
    lo = jnp.searchsorted(sxs, tb[:, 0] - WINDOW, side="left")
    hi = jnp.searchsorted(sxs, tb[:, -1] + WINDOW, side="right")
    clo = (lo // SBLK).astype(jnp.int32)
    chi = ((hi + SBLK - 1) // SBLK).astype(jnp.int32)

    out = pl.pallas_call(
        _chamfer_kernel,
        grid=(n_tblk,),
        in_specs=[
            pl.BlockSpec(memory_space=pltpu.SMEM),
            pl.BlockSpec(memory_space=pltpu.SMEM),
            pl.BlockSpec((TBLK, 3), lambda i: (i, 0)),
            pl.BlockSpec((n_schunk, 3, SBLK), lambda i: (0, 0, 0)),
        ],
        out_specs=pl.BlockSpec((1, 1), lambda i: (0, 0)),
        out_shape=jax.ShapeDtypeStruct((1, 1), jnp.float32),
    )(clo, chi, temp_s, scan_s)
    return out[0, 0]
